# baseline (speedup 1.0000x reference)
import functools

import jax
import jax.numpy as jnp
from jax import lax
from jax.experimental import pallas as pl
from jax.experimental.pallas import tpu as pltpu
from jax.experimental.pallas import tpu_sc as plsc

NUM_LAYERS = 12
N_TASKS = 1000
NUM_PROMPTS = 20
EMB_D = 128
BATCH = 1024
D = NUM_PROMPTS * EMB_D

NC = 2
NS = 16
NW = NC * NS
BPW = BATCH // NW
CHUNK = 16


def _sc_body(table, task, out, idx_all, idx_g0, idx_g1, rows0, rows1,
             sg0, sg1, sw0, sw1):
    wid = lax.axis_index("s") * NC + lax.axis_index("c")
    base = wid * BPW
    pltpu.sync_copy(task.at[pl.ds(base, BPW)], idx_all)

    idx_g = (idx_g0, idx_g1)
    rows = (rows0, rows1)
    sg = (sg0, sg1)
    sw = (sw0, sw1)

    def prep_and_fire(l, j, b):
        ids = idx_all[pl.ds(j * CHUNK, CHUNK)]
        idx_g[b][...] = ids + l * N_TASKS
        pltpu.async_copy(table.at[idx_g[b]], rows[b], sg[b])

    def wait_gather(b):
        pltpu.make_async_copy(table.at[idx_g[b]], rows[b], sg[b]).wait()

    def wait_write(b):
        pltpu.make_async_copy(rows[b], out.at[pl.ds(0, CHUNK)], sw[b]).wait()

    prep_and_fire(0, 0, 0)

    def layer_step(l, carry):
        @pl.when(l > 0)
        def _():
            wait_write(1)
        prep_and_fire(l, 1, 1)
        wait_gather(0)
        pltpu.async_copy(rows[0], out.at[pl.ds(l * BATCH + base, CHUNK)], sw[0])

        @pl.when(l < NUM_LAYERS - 1)
        def _():
            wait_write(0)
            prep_and_fire(l + 1, 0, 0)
        wait_gather(1)
        pltpu.async_copy(
            rows[1], out.at[pl.ds(l * BATCH + base + CHUNK, CHUNK)], sw[1])
        return carry

    lax.fori_loop(0, NUM_LAYERS, layer_step, 0)

    wait_write(0)
    wait_write(1)


@functools.partial(
    pl.kernel,
    mesh=plsc.VectorSubcoreMesh(core_axis_name="c", subcore_axis_name="s"),
    out_type=jax.ShapeDtypeStruct((NUM_LAYERS * BATCH, NUM_PROMPTS, EMB_D), jnp.float32),
    compiler_params=pltpu.CompilerParams(use_tc_tiling_on_sc=True),
    scratch_types=[
        pltpu.VMEM((BPW,), jnp.int32),
        pltpu.VMEM((CHUNK,), jnp.int32),
        pltpu.VMEM((CHUNK,), jnp.int32),
        pltpu.VMEM((CHUNK, NUM_PROMPTS, EMB_D), jnp.float32),
        pltpu.VMEM((CHUNK, NUM_PROMPTS, EMB_D), jnp.float32),
        pltpu.SemaphoreType.DMA,
        pltpu.SemaphoreType.DMA,
        pltpu.SemaphoreType.DMA,
        pltpu.SemaphoreType.DMA,
    ],
)
def _gather_sc(table, task, out, idx_all, idx_g0, idx_g1, rows0, rows1,
               sg0, sg1, sw0, sw1):
    _sc_body(table, task, out, idx_all, idx_g0, idx_g1, rows0, rows1,
             sg0, sg1, sw0, sw1)


def kernel(nL, task_id, e_p):
    table = e_p.reshape(NUM_LAYERS * N_TASKS, NUM_PROMPTS, EMB_D)
    out = _gather_sc(table, task_id)
    return out.reshape(NUM_LAYERS, BATCH, NUM_PROMPTS, EMB_D)


# speedup vs baseline: 2.6443x; 2.6443x over previous
import functools

import jax
import jax.numpy as jnp
from jax import lax
from jax.experimental import pallas as pl
from jax.experimental.pallas import tpu as pltpu
from jax.experimental.pallas import tpu_sc as plsc

NUM_LAYERS = 12
N_TASKS = 1000
NUM_PROMPTS = 20
EMB_D = 128
BATCH = 1024

N_TABLES = NUM_LAYERS * NUM_PROMPTS
NC = 2
NS = 16
NW = NC * NS
BPW = BATCH // NW
TPC = 4
N_CHUNKS = N_TABLES // TPC
ROWS = TPC * BPW


def _sc_body(table, task, out, tid, idx0, idx1, rows0, rows1,
             sg0, sg1, sw0, sw1):
    wid = lax.axis_index("s") * NC + lax.axis_index("c")
    base = wid * BPW
    pltpu.sync_copy(task.at[pl.ds(base, BPW)], tid)
    t0 = tid[pl.ds(0, 16)]
    t1 = tid[pl.ds(16, 16)]

    idx = (idx0, idx1)
    rows = (rows0, rows1)
    sg = (sg0, sg1)
    sw = (sw0, sw1)

    def prep_and_fire(ch, b):
        lp0 = ch * TPC
        for j in range(TPC):
            off = (lp0 + j) * N_TASKS
            idx[b][pl.ds(2 * j * 16, 16)] = t0 + off
            idx[b][pl.ds((2 * j + 1) * 16, 16)] = t1 + off
        pltpu.async_copy(table.at[idx[b]], rows[b], sg[b])

    def wait_gather(b):
        pltpu.make_async_copy(table.at[idx[b]], rows[b], sg[b]).wait()

    def fire_writes(ch, b):
        lp0 = ch * TPC
        for j in range(TPC):
            pltpu.async_copy(
                rows[b].at[pl.ds(j * BPW, BPW)],
                out.at[pl.ds((lp0 + j) * BATCH + base, BPW)],
                sw[b])

    def wait_writes(b):
        for _ in range(TPC):
            pltpu.make_async_copy(
                rows[b].at[pl.ds(0, BPW)], out.at[pl.ds(0, BPW)],
                sw[b]).wait()

    prep_and_fire(0, 0)

    def step(c, carry):
        ch0 = 2 * c
        @pl.when(c > 0)
        def _():
            wait_writes(1)
        prep_and_fire(ch0 + 1, 1)
        wait_gather(0)
        fire_writes(ch0, 0)

        @pl.when(c < N_CHUNKS // 2 - 1)
        def _():
            wait_writes(0)
            prep_and_fire(ch0 + 2, 0)
        wait_gather(1)
        fire_writes(ch0 + 1, 1)
        return carry

    lax.fori_loop(0, N_CHUNKS // 2, step, 0)

    wait_writes(0)
    wait_writes(1)


@functools.partial(
    pl.kernel,
    mesh=plsc.VectorSubcoreMesh(core_axis_name="c", subcore_axis_name="s"),
    out_type=jax.ShapeDtypeStruct((N_TABLES * BATCH, EMB_D), jnp.float32),
    scratch_types=[
        pltpu.VMEM((BPW,), jnp.int32),
        pltpu.VMEM((ROWS,), jnp.int32),
        pltpu.VMEM((ROWS,), jnp.int32),
        pltpu.VMEM((ROWS, EMB_D), jnp.float32),
        pltpu.VMEM((ROWS, EMB_D), jnp.float32),
        pltpu.SemaphoreType.DMA,
        pltpu.SemaphoreType.DMA,
        pltpu.SemaphoreType.DMA,
        pltpu.SemaphoreType.DMA,
    ],
)
def _gather_sc(table, task, out, tid, idx0, idx1, rows0, rows1,
               sg0, sg1, sw0, sw1):
    _sc_body(table, task, out, tid, idx0, idx1, rows0, rows1,
             sg0, sg1, sw0, sw1)


def kernel(nL, task_id, e_p):
    table = e_p.transpose(0, 2, 1, 3).reshape(N_TABLES * N_TASKS, EMB_D)
    out = _gather_sc(table, task_id)
    out = out.reshape(NUM_LAYERS, NUM_PROMPTS, BATCH, EMB_D)
    return out.transpose(0, 2, 1, 3)


# speedup vs baseline: 2.7438x; 1.0376x over previous
import functools

import jax
import jax.numpy as jnp
from jax import lax
from jax.experimental import pallas as pl
from jax.experimental.pallas import tpu as pltpu
from jax.experimental.pallas import tpu_sc as plsc

NUM_LAYERS = 12
N_TASKS = 1000
NUM_PROMPTS = 20
EMB_D = 128
BATCH = 1024

N_TABLES = NUM_LAYERS * NUM_PROMPTS
NC = 2
NS = 16
NW = NC * NS
BPW = BATCH // NW
TPC = 4
N_CHUNKS = N_TABLES // TPC
ROWS = TPC * BPW


NBUF = 4


def _sc_body(table, task, out, tid, idx, rows, sg, sw):
    wid = lax.axis_index("s") * NC + lax.axis_index("c")
    base = wid * BPW
    pltpu.sync_copy(task.at[pl.ds(base, BPW)], tid)
    t0 = tid[pl.ds(0, 16)]
    t1 = tid[pl.ds(16, 16)]

    def prep_and_fire(ch, b):
        lp0 = ch * TPC
        for j in range(TPC):
            off = (lp0 + j) * N_TASKS
            idx[b][pl.ds(2 * j * 16, 16)] = t0 + off
            idx[b][pl.ds((2 * j + 1) * 16, 16)] = t1 + off
        pltpu.async_copy(table.at[idx[b]], rows[b], sg[b])

    def wait_gather(b):
        pltpu.make_async_copy(table.at[idx[b]], rows[b], sg[b]).wait()

    def fire_writes(ch, b):
        lp0 = ch * TPC
        for j in range(TPC):
            pltpu.async_copy(
                rows[b].at[pl.ds(j * BPW, BPW)],
                out.at[pl.ds((lp0 + j) * BATCH + base, BPW)],
                sw[b])

    def wait_writes(b):
        for _ in range(TPC):
            pltpu.make_async_copy(
                rows[b].at[pl.ds(0, BPW)], out.at[pl.ds(0, BPW)],
                sw[b]).wait()

    for b in range(NBUF - 1):
        prep_and_fire(b, b)

    n_steps = N_CHUNKS // NBUF

    def step(c, carry):
        for b in range(NBUF):
            ch = NBUF * c + b
            nb = (b + NBUF - 1) % NBUF
            if b == 0:
                @pl.when(c > 0)
                def _():
                    wait_writes(nb)
                prep_and_fire(ch + NBUF - 1, nb)
            else:
                @pl.when(c < n_steps - 1)
                def _():
                    wait_writes(nb)
                    prep_and_fire(ch + NBUF - 1, nb)
            wait_gather(b)
            fire_writes(ch, b)
        return carry

    lax.fori_loop(0, n_steps, step, 0)

    for b in range(NBUF):
        wait_writes(b)


@functools.partial(
    pl.kernel,
    mesh=plsc.VectorSubcoreMesh(core_axis_name="c", subcore_axis_name="s"),
    out_type=jax.ShapeDtypeStruct((N_TABLES * BATCH, EMB_D), jnp.float32),
    scratch_types=(
        [pltpu.VMEM((BPW,), jnp.int32)]
        + [pltpu.VMEM((ROWS,), jnp.int32)] * NBUF
        + [pltpu.VMEM((ROWS, EMB_D), jnp.float32)] * NBUF
        + [pltpu.SemaphoreType.DMA] * (2 * NBUF)
    ),
)
def _gather_sc(table, task, out, tid, *scratch):
    idx = scratch[:NBUF]
    rows = scratch[NBUF:2 * NBUF]
    sg = scratch[2 * NBUF:3 * NBUF]
    sw = scratch[3 * NBUF:]
    _sc_body(table, task, out, tid, idx, rows, sg, sw)


def kernel(nL, task_id, e_p):
    table = e_p.transpose(0, 2, 1, 3).reshape(N_TABLES * N_TASKS, EMB_D)
    out = _gather_sc(table, task_id)
    out = out.reshape(NUM_LAYERS, NUM_PROMPTS, BATCH, EMB_D)
    return out.transpose(0, 2, 1, 3)


# speedup vs baseline: 2.7598x; 1.0058x over previous
import functools

import jax
import jax.numpy as jnp
from jax import lax
from jax.experimental import pallas as pl
from jax.experimental.pallas import tpu as pltpu
from jax.experimental.pallas import tpu_sc as plsc

NUM_LAYERS = 12
N_TASKS = 1000
NUM_PROMPTS = 20
EMB_D = 128
BATCH = 1024

N_TABLES = NUM_LAYERS * NUM_PROMPTS
NC = 2
NS = 16
NW = NC * NS
BPW = BATCH // NW
TPC = 4
N_CHUNKS = N_TABLES // TPC
ROWS = TPC * BPW


NBUF = 6


def _sc_body(table, task, out, tid, idx, rows, sg, sw):
    wid = lax.axis_index("s") * NC + lax.axis_index("c")
    base = wid * BPW
    pltpu.sync_copy(task.at[pl.ds(base, BPW)], tid)
    t0 = tid[pl.ds(0, 16)]
    t1 = tid[pl.ds(16, 16)]

    def prep_and_fire(ch, b):
        lp0 = ch * TPC
        for j in range(TPC):
            off = (lp0 + j) * N_TASKS
            idx[b][pl.ds(2 * j * 16, 16)] = t0 + off
            idx[b][pl.ds((2 * j + 1) * 16, 16)] = t1 + off
        pltpu.async_copy(table.at[idx[b]], rows[b], sg[b])

    def wait_gather(b):
        pltpu.make_async_copy(table.at[idx[b]], rows[b], sg[b]).wait()

    def fire_writes(ch, b):
        lp0 = ch * TPC
        for j in range(TPC):
            pltpu.async_copy(
                rows[b].at[pl.ds(j * BPW, BPW)],
                out.at[pl.ds((lp0 + j) * BATCH + base, BPW)],
                sw[b])

    def wait_writes(b):
        for _ in range(TPC):
            pltpu.make_async_copy(
                rows[b].at[pl.ds(0, BPW)], out.at[pl.ds(0, BPW)],
                sw[b]).wait()

    for b in range(NBUF - 1):
        prep_and_fire(b, b)

    n_steps = N_CHUNKS // NBUF

    def step(c, carry):
        for b in range(NBUF):
            ch = NBUF * c + b
            nb = (b + NBUF - 1) % NBUF
            if b == 0:
                @pl.when(c > 0)
                def _():
                    wait_writes(nb)
                prep_and_fire(ch + NBUF - 1, nb)
            else:
                @pl.when(c < n_steps - 1)
                def _():
                    wait_writes(nb)
                    prep_and_fire(ch + NBUF - 1, nb)
            wait_gather(b)
            fire_writes(ch, b)
        return carry

    lax.fori_loop(0, n_steps, step, 0)

    for b in range(NBUF):
        wait_writes(b)


@functools.partial(
    pl.kernel,
    mesh=plsc.VectorSubcoreMesh(core_axis_name="c", subcore_axis_name="s"),
    out_type=jax.ShapeDtypeStruct((N_TABLES * BATCH, EMB_D), jnp.float32),
    scratch_types=(
        [pltpu.VMEM((BPW,), jnp.int32)]
        + [pltpu.VMEM((ROWS,), jnp.int32)] * NBUF
        + [pltpu.VMEM((ROWS, EMB_D), jnp.float32)] * NBUF
        + [pltpu.SemaphoreType.DMA] * (2 * NBUF)
    ),
)
def _gather_sc(table, task, out, tid, *scratch):
    idx = scratch[:NBUF]
    rows = scratch[NBUF:2 * NBUF]
    sg = scratch[2 * NBUF:3 * NBUF]
    sw = scratch[3 * NBUF:]
    _sc_body(table, task, out, tid, idx, rows, sg, sw)


def kernel(nL, task_id, e_p):
    table = e_p.transpose(0, 2, 1, 3).reshape(N_TABLES * N_TASKS, EMB_D)
    out = _gather_sc(table, task_id)
    out = out.reshape(NUM_LAYERS, NUM_PROMPTS, BATCH, EMB_D)
    return out.transpose(0, 2, 1, 3)


# speedup vs baseline: 2.7891x; 1.0106x over previous
import functools

import jax
import jax.numpy as jnp
from jax import lax
from jax.experimental import pallas as pl
from jax.experimental.pallas import tpu as pltpu
from jax.experimental.pallas import tpu_sc as plsc

NUM_LAYERS = 12
N_TASKS = 1000
NUM_PROMPTS = 20
EMB_D = 128
BATCH = 1024

N_TABLES = NUM_LAYERS * NUM_PROMPTS
NC = 2
NS = 16
NW = NC * NS
BPW = BATCH // NW
TPC = 4
N_CHUNKS = N_TABLES // TPC
ROWS = TPC * BPW


NBUF = 6


def _sc_body(table, task, out, tid, idx, rows, sg, sw):
    wid = lax.axis_index("s") * NC + lax.axis_index("c")
    base = wid * BPW
    pltpu.sync_copy(task.at[pl.ds(base, BPW)], tid)
    t0 = tid[pl.ds(0, 16)]
    t1 = tid[pl.ds(16, 16)]

    ch_off = (wid * N_CHUNKS) // NW

    def prep_and_fire(ch, b):
        ch = lax.rem(ch + ch_off, N_CHUNKS)
        lp0 = ch * TPC
        for j in range(TPC):
            off = (lp0 + j) * N_TASKS
            idx[b][pl.ds(2 * j * 16, 16)] = t0 + off
            idx[b][pl.ds((2 * j + 1) * 16, 16)] = t1 + off
        pltpu.async_copy(table.at[idx[b]], rows[b], sg[b])

    def wait_gather(b):
        pltpu.make_async_copy(table.at[idx[b]], rows[b], sg[b]).wait()

    def fire_writes(ch, b):
        ch = lax.rem(ch + ch_off, N_CHUNKS)
        lp0 = ch * TPC
        for j in range(TPC):
            pltpu.async_copy(
                rows[b].at[pl.ds(j * BPW, BPW)],
                out.at[pl.ds((lp0 + j) * BATCH + base, BPW)],
                sw[b])

    def wait_writes(b):
        for _ in range(TPC):
            pltpu.make_async_copy(
                rows[b].at[pl.ds(0, BPW)], out.at[pl.ds(0, BPW)],
                sw[b]).wait()

    for b in range(NBUF - 1):
        prep_and_fire(b, b)

    n_steps = N_CHUNKS // NBUF

    def step(c, carry):
        for b in range(NBUF):
            ch = NBUF * c + b
            nb = (b + NBUF - 1) % NBUF
            if b == 0:
                @pl.when(c > 0)
                def _():
                    wait_writes(nb)
                prep_and_fire(ch + NBUF - 1, nb)
            else:
                @pl.when(c < n_steps - 1)
                def _():
                    wait_writes(nb)
                    prep_and_fire(ch + NBUF - 1, nb)
            wait_gather(b)
            fire_writes(ch, b)
        return carry

    lax.fori_loop(0, n_steps, step, 0)

    for b in range(NBUF):
        wait_writes(b)


@functools.partial(
    pl.kernel,
    mesh=plsc.VectorSubcoreMesh(core_axis_name="c", subcore_axis_name="s"),
    out_type=jax.ShapeDtypeStruct((N_TABLES * BATCH, EMB_D), jnp.float32),
    scratch_types=(
        [pltpu.VMEM((BPW,), jnp.int32)]
        + [pltpu.VMEM((ROWS,), jnp.int32)] * NBUF
        + [pltpu.VMEM((ROWS, EMB_D), jnp.float32)] * NBUF
        + [pltpu.SemaphoreType.DMA] * (2 * NBUF)
    ),
)
def _gather_sc(table, task, out, tid, *scratch):
    idx = scratch[:NBUF]
    rows = scratch[NBUF:2 * NBUF]
    sg = scratch[2 * NBUF:3 * NBUF]
    sw = scratch[3 * NBUF:]
    _sc_body(table, task, out, tid, idx, rows, sg, sw)


def kernel(nL, task_id, e_p):
    table = e_p.transpose(0, 2, 1, 3).reshape(N_TABLES * N_TASKS, EMB_D)
    out = _gather_sc(table, task_id)
    out = out.reshape(NUM_LAYERS, NUM_PROMPTS, BATCH, EMB_D)
    return out.transpose(0, 2, 1, 3)
